# baseline (speedup 1.0000x reference)
import functools

import jax
import jax.numpy as jnp
import numpy as np
from jax import lax
from jax.experimental import pallas as pl
from jax.experimental.pallas import tpu as pltpu
from jax.experimental.pallas import tpu_sc as plsc

_FIELD_DIMS = [100000] * 26
_NF = len(_FIELD_DIMS)
_ED = 16
_B = 16384
_TOTAL_ROWS = int(sum(_FIELD_DIMS))
_OFFS = np.concatenate(([0], np.cumsum(_FIELD_DIMS)[:-1])).astype(np.int32)

_NC = 2
_NS = 16
_NW = _NC * _NS
_BPW = _B // _NW
_C = 128
_NCH = _BPW // _C
_G = _C // 16

_mesh = plsc.VectorSubcoreMesh(
    core_axis_name="c", subcore_axis_name="s", num_cores=_NC, num_subcores=_NS
)


@functools.partial(
    pl.kernel,
    out_type=jax.ShapeDtypeStruct((_B,), jnp.float32),
    mesh=_mesh,
    compiler_params=pltpu.CompilerParams(
        needs_layout_passes=False, use_tc_tiling_on_sc=False
    ),
    scratch_types=[
        pltpu.VMEM((_C, _NF), jnp.int32),
        pltpu.VMEM((_C, _NF), jnp.int32),
        pltpu.VMEM((_NF, _C), jnp.int32),
        pltpu.VMEM((_NF, _C), jnp.int32),
        pltpu.VMEM((_NF, _C), jnp.int32),
        pltpu.VMEM((_NF, _C), jnp.int32),
        pltpu.VMEM((_NF * _C, _ED), jnp.float32),
        pltpu.VMEM((_C,), jnp.float32),
        pltpu.VMEM((_C,), jnp.float32),
        pltpu.VMEM((_BPW,), jnp.float32),
        pltpu.VMEM((16,), jnp.float32),
        pltpu.SemaphoreType.DMA,
    ],
)
def _sc_fm(li_hbm, fi_hbm, emb_hbm, lin2d_hbm, bias_hbm, out_hbm,
           li_raw, fi_raw, li_gl, fi_gl, li_q, fi_q, rows, s1buf, s2buf,
           obuf, bias_v, sem):
    wid = lax.axis_index("s") * _NC + lax.axis_index("c")
    base = wid * _BPW

    pltpu.sync_copy(bias_hbm, bias_v)
    bvec = bias_v[...]
    lanes = lax.iota(jnp.int32, 16)

    def chunk_body(ch, carry):
        cbase = base + ch * _C
        cp1 = pltpu.async_copy(li_hbm.at[pl.ds(cbase, _C), :], li_raw, sem)
        cp2 = pltpu.async_copy(fi_hbm.at[pl.ds(cbase, _C), :], fi_raw, sem)
        cp1.wait()
        cp2.wait()

        def tr_body(g, c):
            r = lanes + g * 16
            sl = pl.ds(g * 16, 16)
            for f in range(_NF):
                cvec = jnp.full((16,), f, jnp.int32)
                lg = plsc.load_gather(li_raw, [r, cvec]) + _OFFS[f]
                fg = plsc.load_gather(fi_raw, [r, cvec]) + _OFFS[f]
                li_gl[f, sl] = lg
                fi_gl[f, sl] = fg
                li_q[f, sl] = lax.shift_right_logical(lg, 4)
                fi_q[f, sl] = lax.shift_right_logical(fg, 4)
            return c

        lax.fori_loop(0, _G, tr_body, 0)

        mask15 = jnp.full((16,), 15, jnp.int32)

        def lin_pass(q_ref, gl_ref, sbuf):
            copies = []
            for f in range(_NF):
                copies.append(pltpu.async_copy(
                    lin2d_hbm.at[q_ref.at[f]], rows.at[pl.ds(f * _C, _C)], sem))
            for cp in copies:
                cp.wait()

            def red_body(g, c):
                bidx = lanes + g * 16
                sl = pl.ds(g * 16, 16)
                acc = jnp.zeros((16,), jnp.float32)
                for f in range(_NF):
                    rem = lax.bitwise_and(gl_ref[f, sl], mask15)
                    acc = acc + plsc.load_gather(rows, [bidx + f * _C, rem])
                sbuf[sl] = acc
                return c

            lax.fori_loop(0, _G, red_body, 0)

        lin_pass(li_q, li_gl, s1buf)
        lin_pass(fi_q, fi_gl, s2buf)

        copies = []
        for f in range(_NF):
            copies.append(pltpu.async_copy(
                emb_hbm.at[fi_gl.at[f]], rows.at[pl.ds(f * _C, _C)], sem))
        for cp in copies:
            cp.wait()

        def grp_body(g, c):
            bidx = lanes + g * 16
            sl = pl.ds(g * 16, 16)
            ss = jnp.zeros((16,), jnp.float32)
            fmacc = jnp.zeros((16,), jnp.float32)
            for d in range(_ED):
                dvec = jnp.full((16,), d, jnp.int32)
                sd = jnp.zeros((16,), jnp.float32)
                for f in range(_NF):
                    v = plsc.load_gather(rows, [bidx + f * _C, dvec])
                    sd = sd + v
                    ss = ss + v * v
                fmacc = fmacc + sd * sd
            x = s1buf[sl] + s2buf[sl] + 2.0 * bvec + 0.5 * (fmacc - ss)
            y = 1.0 / (1.0 + jnp.exp(-x))
            obuf[pl.ds(ch * _C + g * 16, 16)] = y
            return c

        lax.fori_loop(0, _G, grp_body, 0)
        return carry

    lax.fori_loop(0, _NCH, chunk_body, 0)
    pltpu.sync_copy(obuf, out_hbm.at[pl.ds(base, _BPW)])


def kernel(linear_x, fm_x, emb_table, lin_table, lin_bias):
    li = linear_x.astype(jnp.int32)
    fi = fm_x.astype(jnp.int32)
    lin2d = lin_table.reshape(_TOTAL_ROWS // 16, 16)
    bias16 = jnp.broadcast_to(lin_bias.astype(jnp.float32), (16,))
    return _sc_fm(li, fi, emb_table, lin2d, bias16)
